# baseline (speedup 1.0000x reference)
import functools

import jax
import jax.numpy as jnp
from jax import lax
from jax.experimental import pallas as pl
from jax.experimental.pallas import tpu as pltpu
from jax.experimental.pallas import tpu_sc as plsc

VOCAB = 25
EMBED_DIM = 32

_ROWS = 16384
_COLS = 200
_B = _ROWS * _COLS

_NC = 2
_NS = 16
_NW = _NC * _NS
_B_PER_W = _B // _NW

_CH = 1600
_N_CHUNKS = _B_PER_W // _CH
_N_OUTER = _N_CHUNKS // 2

_mesh = plsc.VectorSubcoreMesh(core_axis_name="c", subcore_axis_name="s")


@functools.partial(
    pl.kernel,
    mesh=_mesh,
    out_type=jax.ShapeDtypeStruct((_B, EMBED_DIM), jnp.float32),
    scratch_types=[
        pltpu.VMEM((_CH,), jnp.int32),
        pltpu.VMEM((_CH,), jnp.int32),
        pltpu.VMEM((_CH, EMBED_DIM), jnp.float32),
        pltpu.VMEM((_CH, EMBED_DIM), jnp.float32),
        pltpu.SemaphoreType.DMA,
        pltpu.SemaphoreType.DMA,
        pltpu.SemaphoreType.DMA,
        pltpu.SemaphoreType.DMA,
        pltpu.SemaphoreType.DMA,
        pltpu.SemaphoreType.DMA,
    ],
    compiler_params=pltpu.CompilerParams(use_tc_tiling_on_sc=False),
)
def _gather_kernel(table_hbm, idx_hbm, out_hbm,
                   idx_v0, idx_v1, rows_v0, rows_v1,
                   si0, si1, sg0, sg1, so0, so1):
    wid = lax.axis_index("s") * _NC + lax.axis_index("c")
    base = wid * _B_PER_W
    idx_v = (idx_v0, idx_v1)
    rows_v = (rows_v0, rows_v1)
    si = (si0, si1)
    sg = (sg0, sg1)
    so = (so0, so1)

    for b in range(2):
        pltpu.async_copy(idx_hbm.at[pl.ds(base + b * _CH, _CH)], idx_v[b], si[b])

    def outer(gg, carry):
        for b in range(2):
            g = gg * 2 + b
            off = base + g * _CH

            pltpu.make_async_copy(
                idx_hbm.at[pl.ds(0, _CH)], idx_v[b], si[b]).wait()

            @pl.when(gg >= 1)
            def _(b=b):
                pltpu.make_async_copy(
                    rows_v[b], out_hbm.at[pl.ds(0, _CH)], so[b]).wait()

            pltpu.async_copy(table_hbm.at[idx_v[b]], rows_v[b], sg[b]).wait()

            @pl.when(gg < _N_OUTER - 1)
            def _(b=b, off=off):
                pltpu.async_copy(
                    idx_hbm.at[pl.ds(off + 2 * _CH, _CH)], idx_v[b], si[b])

            pltpu.async_copy(rows_v[b], out_hbm.at[pl.ds(off, _CH)], so[b])
        return carry

    lax.fori_loop(0, _N_OUTER, outer, 0)

    for b in range(2):
        pltpu.make_async_copy(
            rows_v[b], out_hbm.at[pl.ds(0, _CH)], so[b]).wait()


def kernel(x, table):
    idx = x.reshape(_B).astype(jnp.int32)
    out = _gather_kernel(table, idx)
    return out.reshape(_ROWS, _COLS, EMBED_DIM)


# speedup vs baseline: 3.9687x; 3.9687x over previous
import functools

import jax
import jax.numpy as jnp
from jax import lax
from jax.experimental import pallas as pl
from jax.experimental.pallas import tpu as pltpu
from jax.experimental.pallas import tpu_sc as plsc

VOCAB = 25
EMBED_DIM = 32

_ROWS = 16384
_COLS = 200
_B = _ROWS * _COLS

_NC = 2
_NS = 16
_NW = _NC * _NS
_B_PER_W = _B // _NW

_CH = 1600
_N_CHUNKS = _B_PER_W // _CH
_N_OUTER = _N_CHUNKS // 2

_mesh = plsc.VectorSubcoreMesh(core_axis_name="c", subcore_axis_name="s")


@functools.partial(
    pl.kernel,
    mesh=_mesh,
    out_type=jax.ShapeDtypeStruct((_B, EMBED_DIM), jnp.float32),
    scratch_types=[
        pltpu.VMEM_SHARED((VOCAB, EMBED_DIM), jnp.float32),
        pltpu.VMEM((_CH,), jnp.int32),
        pltpu.VMEM((_CH,), jnp.int32),
        pltpu.VMEM((_CH, EMBED_DIM), jnp.float32),
        pltpu.VMEM((_CH, EMBED_DIM), jnp.float32),
        pltpu.SemaphoreType.DMA,
        pltpu.SemaphoreType.DMA,
        pltpu.SemaphoreType.DMA,
        pltpu.SemaphoreType.DMA,
        pltpu.SemaphoreType.DMA,
        pltpu.SemaphoreType.DMA,
    ],
    compiler_params=pltpu.CompilerParams(use_tc_tiling_on_sc=False),
)
def _gather_kernel(table_hbm, idx_hbm, out_hbm,
                   table_v, idx_v0, idx_v1, rows_v0, rows_v1,
                   si0, si1, sg0, sg1, so0, so1):
    wid = lax.axis_index("s") * _NC + lax.axis_index("c")
    base = wid * _B_PER_W

    @pl.when(lax.axis_index("s") == 0)
    def _():
        pltpu.sync_copy(table_hbm, table_v)
    plsc.subcore_barrier()
    idx_v = (idx_v0, idx_v1)
    rows_v = (rows_v0, rows_v1)
    si = (si0, si1)
    sg = (sg0, sg1)
    so = (so0, so1)

    for b in range(2):
        pltpu.async_copy(idx_hbm.at[pl.ds(base + b * _CH, _CH)], idx_v[b], si[b])

    def outer(gg, carry):
        for b in range(2):
            g = gg * 2 + b
            off = base + g * _CH

            pltpu.make_async_copy(
                idx_hbm.at[pl.ds(0, _CH)], idx_v[b], si[b]).wait()

            @pl.when(gg >= 1)
            def _(b=b):
                pltpu.make_async_copy(
                    rows_v[b], out_hbm.at[pl.ds(0, _CH)], so[b]).wait()

            pltpu.async_copy(table_v.at[idx_v[b]], rows_v[b], sg[b]).wait()

            @pl.when(gg < _N_OUTER - 1)
            def _(b=b, off=off):
                pltpu.async_copy(
                    idx_hbm.at[pl.ds(off + 2 * _CH, _CH)], idx_v[b], si[b])

            pltpu.async_copy(rows_v[b], out_hbm.at[pl.ds(off, _CH)], so[b])
        return carry

    lax.fori_loop(0, _N_OUTER, outer, 0)

    for b in range(2):
        pltpu.make_async_copy(
            rows_v[b], out_hbm.at[pl.ds(0, _CH)], so[b]).wait()


def kernel(x, table):
    idx = x.reshape(_B).astype(jnp.int32)
    out = _gather_kernel(table, idx)
    return out.reshape(_ROWS, _COLS, EMBED_DIM)
